# baseline (speedup 1.0000x reference)
import functools

import jax
import jax.numpy as jnp
from jax import lax
from jax.experimental import pallas as pl
from jax.experimental.pallas import tpu as pltpu
from jax.experimental.pallas import tpu_sc as plsc

NC = 2
NS = 16
NW = NC * NS

G = 128
GPC = 5
CH = G * GPC


@functools.lru_cache(maxsize=None)
def _build(n, d, table_dtype):
    per_w = n // NW
    nch = per_w // CH

    mesh = plsc.VectorSubcoreMesh(core_axis_name="c", subcore_axis_name="s")

    @functools.partial(
        pl.kernel,
        mesh=mesh,
        out_type=jax.ShapeDtypeStruct((n, d), table_dtype),
        scratch_types=[
            pltpu.VMEM((per_w,), jnp.int32),
            pltpu.VMEM((CH, d), table_dtype),
            pltpu.SemaphoreType.DMA,
        ],
        compiler_params=pltpu.CompilerParams(use_tc_tiling_on_sc=False),
    )
    def emb(idx_hbm, table_hbm, out_hbm, idx_v, rows_v, gsem):
        wid = lax.axis_index("s") * NC + lax.axis_index("c")
        base = wid * per_w
        pltpu.sync_copy(idx_hbm.at[pl.ds(base, per_w)], idx_v)

        def chunk(c, carry):
            off = c * CH
            handles = [
                pltpu.async_copy(
                    table_hbm.at[idx_v.at[pl.ds(off + g * G, G)]],
                    rows_v.at[pl.ds(g * G, G)],
                    gsem,
                )
                for g in range(GPC)
            ]
            for h in handles:
                h.wait()
            pltpu.sync_copy(rows_v, out_hbm.at[pl.ds(base + off, CH)])
            return carry

        lax.fori_loop(0, nch, chunk, 0)

    return emb


def kernel(indices, table):
    b, s = indices.shape
    d = table.shape[1]
    flat = indices.reshape(-1)
    emb = _build(flat.shape[0], d, table.dtype)
    out = emb(flat, table)
    return out.reshape(b, s, d)
